# Initial kernel scaffold; baseline (speedup 1.0000x reference)
import jax
import jax.numpy as jnp
from jax.experimental import pallas as pl


def kernel(x_user, x_item, ei_user_item, ei_item_user, Win, bin_, Wk, Wq, Wv, Wa, skip, arel, mrel, prel):
    raise NotImplementedError("write your pallas kernel here")


# speedup vs baseline: 1.4718x; 1.4718x over previous
import functools

import jax
import jax.numpy as jnp
from jax import lax
from jax.experimental import pallas as pl
from jax.experimental.pallas import tpu as pltpu

H = 4
D = 32
HID = 128
L = 2
T = 2
R = 2
N = 50000
E = 300000
ETYPES = ((0, 1), (1, 0))

_BM = 2000


def _mm_kernel(x_ref, w_ref, o_ref, *, mode, b_ref=None, res_ref=None, beta_ref=None):
    x = x_ref[...]
    if mode == "gelu_pre":
        x = jax.nn.gelu(x)
    z = jnp.dot(x, w_ref[...], preferred_element_type=jnp.float32)
    if b_ref is not None:
        z = z + b_ref[...]
    if mode == "relu":
        z = jax.nn.relu(z)
    if res_ref is not None:
        beta = beta_ref[0]
        z = beta * z + (1.0 - beta) * res_ref[...]
    o_ref[...] = z


def _tc_matmul(x, w, b=None, res=None, beta=None, mode="none"):
    m, k = x.shape
    n = w.shape[1]
    grid = (m // _BM,)
    in_specs = [
        pl.BlockSpec((_BM, k), lambda i: (i, 0)),
        pl.BlockSpec((k, n), lambda i: (0, 0)),
    ]
    args = [x, w]
    kw = {}
    if b is not None:
        in_specs.append(pl.BlockSpec((1, n), lambda i: (0, 0)))
        args.append(b.reshape(1, n))
    if res is not None:
        in_specs.append(pl.BlockSpec((_BM, n), lambda i: (i, 0)))
        args.append(res)
        in_specs.append(pl.BlockSpec(memory_space=pltpu.SMEM))
        args.append(beta.reshape(1))
    names = ["b_ref", "res_ref", "beta_ref"]
    sel = [b is not None, res is not None, res is not None]
    body = functools.partial(_mm_kernel, mode=mode)

    def kern2(*refs):
        x_ref, w_ref = refs[0], refs[1]
        idx = 2
        kwargs = {}
        for name, on in zip(names, sel):
            if on:
                kwargs[name] = refs[idx]
                idx += 1
        o_ref = refs[idx]
        body(x_ref, w_ref, o_ref, **kwargs)

    return pl.pallas_call(
        kern2,
        grid=grid,
        in_specs=in_specs,
        out_specs=pl.BlockSpec((_BM, n), lambda i: (i, 0)),
        out_shape=jax.ShapeDtypeStruct((m, n), jnp.float32),
    )(*args)


def _fold_weights(Wk, Wv, arel, mrel, l, r, s_t):
    wk = Wk[l, s_t].reshape(HID, H, D)
    wfk = jnp.einsum("ihd,hde->ihe", wk, arel[l, r]).reshape(HID, HID)
    wv = Wv[l, s_t].reshape(HID, H, D)
    wfv = jnp.einsum("ihd,hde->ihe", wv, mrel[l, r]).reshape(HID, HID)
    return wfk, wfv


def kernel(x_user, x_item, ei_user_item, ei_item_user, Win, bin_, Wk, Wq, Wv, Wa, skip, arel, mrel, prel):
    scale = 1.0 / jnp.sqrt(jnp.float32(D))
    xs = [
        _tc_matmul(x_user, Win[0], b=bin_[0], mode="relu"),
        _tc_matmul(x_item, Win[1], b=bin_[1], mode="relu"),
    ]
    eis = (ei_user_item, ei_item_user)
    for l in range(L):
        zs = []
        for t in range(T):
            r = 0 if ETYPES[0][0] == t else 1
            wfk, wfv = _fold_weights(Wk, Wv, arel, mrel, l, r, t)
            wcat = jnp.concatenate([Wq[l, t], wfk, wfv], axis=1)
            zs.append(_tc_matmul(xs[t], wcat))
        aggn = [None, None]
        for r, (s_t, d_t) in enumerate(ETYPES):
            src = eis[r][0]
            dst = eis[r][1]
            q = zs[d_t][:, 0:HID]
            krel = zs[s_t][:, HID : 2 * HID]
            vrel = zs[s_t][:, 2 * HID : 3 * HID]
            qe = q[dst].reshape(E, H, D)
            ke = krel[src].reshape(E, H, D)
            alpha = jnp.sum(qe * ke, axis=-1) * prel[l, r] * scale
            g = jnp.exp(alpha)
            den = jax.ops.segment_sum(g, dst, num_segments=N)
            ve = vrel[src].reshape(E, H, D)
            num = jax.ops.segment_sum(g[:, :, None] * ve, dst, num_segments=N)
            aggn[d_t] = (num / jnp.maximum(den, 1e-30)[:, :, None]).reshape(N, HID)
        new_xs = []
        for t in range(T):
            beta = jax.nn.sigmoid(skip[l, t])
            new_xs.append(
                _tc_matmul(aggn[t], Wa[l, t], res=xs[t], beta=beta, mode="gelu_pre")
            )
        xs = new_xs
    return jnp.concatenate(xs, axis=0)
